# Initial kernel scaffold; baseline (speedup 1.0000x reference)
import jax
import jax.numpy as jnp
from jax.experimental import pallas as pl


def kernel(seq_len, relative_positions_weight):
    raise NotImplementedError("write your pallas kernel here")


# speedup vs baseline: 5.4770x; 5.4770x over previous
import functools

import jax
import jax.numpy as jnp
from jax import lax
from jax.experimental import pallas as pl
from jax.experimental.pallas import tpu as pltpu
from jax.experimental.pallas import tpu_sc as plsc


@functools.lru_cache(maxsize=None)
def _build_expand(SL: int, D: int):
    info = plsc.get_sparse_core_info()
    NC, NS = info.num_cores, info.num_subcores
    NW = NC * NS
    assert SL % NW == 0
    RPW = SL // NW
    SPAN = SL + RPW
    ROW = SL * D

    mesh = plsc.VectorSubcoreMesh(core_axis_name="c", subcore_axis_name="s")

    @functools.partial(
        pl.kernel,
        mesh=mesh,
        out_type=jax.ShapeDtypeStruct((SL * SL * D,), jnp.float32),
        scratch_types=[
            pltpu.VMEM((SPAN * D,), jnp.float32),
            pltpu.SemaphoreType.DMA,
        ],
    )
    def expand(wrev_hbm, out_hbm, stage, sem):
        wid = lax.axis_index("s") * NC + lax.axis_index("c")
        base = wid * RPW
        lo = (SL - RPW - base) * D
        pltpu.sync_copy(wrev_hbm.at[pl.ds(lo, SPAN * D)], stage)
        copies = []
        for t in range(RPW):
            copies.append(
                pltpu.async_copy(
                    stage.at[pl.ds((RPW - 1 - t) * D, ROW)],
                    out_hbm.at[pl.ds((base + t) * ROW, ROW)],
                    sem,
                )
            )
        for c in copies:
            c.wait()

    return expand


def kernel(seq_len, relative_positions_weight):
    V, D = relative_positions_weight.shape
    SL = (V + 1) // 2
    wrev = jnp.flip(relative_positions_weight, axis=0)
    wrev = jnp.concatenate([wrev, jnp.zeros((1, D), wrev.dtype)], axis=0)
    flat = _build_expand(SL, D)(wrev.reshape(-1))
    return flat.reshape(SL, SL, D)
